# baseline (speedup 1.0000x reference)
import functools

import jax
import jax.numpy as jnp
import numpy as np
from jax import lax
from jax.experimental import pallas as pl
from jax.experimental.pallas import tpu as pltpu
from jax.experimental.pallas import tpu_sc as plsc

_N = 10000
_E = 160000
_G = 16
_D = 120
_H = 4
_L = 64
_SH = 9
_F = 256

_BE = 1024
_EP = 160 * _BE
_DP = 128
_VP = 512


_NC, _NS = 2, 16
_NW = _NC * _NS
_EW = _EP // _NW
_CG = 128


def _gather_body(node_hbm, src_hbm, dst_hbm,
                 gs_hbm, gd_hbm,
                 idx_s0, idx_d0, idx_s1, idx_d1,
                 rows_s, rows_d, sem_g0, sem_g1):
    wid = lax.axis_index("s") * _NC + lax.axis_index("c")
    base = wid * _EW
    nchunk = _EW // _CG
    sems = (sem_g0, sem_g1)
    idx_s = (idx_s0, idx_s1)
    idx_d = (idx_d0, idx_d1)

    def start(c, b):
        off = base + c * _CG
        pltpu.sync_copy(src_hbm.at[pl.ds(off, _CG)], idx_s[b])
        pltpu.sync_copy(dst_hbm.at[pl.ds(off, _CG)], idx_d[b])
        pltpu.async_copy(node_hbm.at[idx_s[b]], rows_s.at[b], sems[b])
        pltpu.async_copy(node_hbm.at[idx_d[b]], rows_d.at[b], sems[b])

    def finish(c, b):
        off = base + c * _CG
        pltpu.make_async_copy(node_hbm.at[idx_s[b]], rows_s.at[b],
                              sems[b]).wait()
        pltpu.make_async_copy(node_hbm.at[idx_d[b]], rows_d.at[b],
                              sems[b]).wait()
        pltpu.sync_copy(rows_s.at[b], gs_hbm.at[pl.ds(off, _CG)])
        pltpu.sync_copy(rows_d.at[b], gd_hbm.at[pl.ds(off, _CG)])

    start(0, 0)

    def step(j, carry):
        c0, c1 = 2 * j, 2 * j + 1
        start(c1, 1)
        finish(c0, 0)

        @pl.when(j < nchunk // 2 - 1)
        def _():
            start(c0 + 2, 0)

        finish(c1, 1)
        return carry

    lax.fori_loop(0, nchunk // 2, step, 0)


def _sc_gather(node_t, src_p, dst_p, width=_DP, tc_tiling=True,
               dtype=jnp.float32):
    f = pl.kernel(
        _gather_body,
        out_type=(jax.ShapeDtypeStruct((_EP, width), dtype),
                  jax.ShapeDtypeStruct((_EP, width), dtype)),
        mesh=plsc.VectorSubcoreMesh(core_axis_name="c", subcore_axis_name="s"),
        scratch_types=[
            pltpu.VMEM((_CG,), jnp.int32),
            pltpu.VMEM((_CG,), jnp.int32),
            pltpu.VMEM((_CG,), jnp.int32),
            pltpu.VMEM((_CG,), jnp.int32),
            pltpu.VMEM((2, _CG, width), dtype),
            pltpu.VMEM((2, _CG, width), dtype),
            pltpu.SemaphoreType.DMA,
            pltpu.SemaphoreType.DMA,
        ],
        compiler_params=pltpu.CompilerParams(use_tc_tiling_on_sc=tc_tiling),
    )
    return f(node_t, src_p, dst_p)


def _geom_body(ps_ref, pd_ref, wa1_ref, ba1_ref, wa2_ref, ba2_ref,
               wa3_ref, ba3_ref, ess_ref, ex_ref):
    diff = pd_ref[...] - ps_ref[...]
    d2 = jnp.sum(diff * diff, axis=-1, keepdims=True)
    dd = jnp.sqrt(d2)
    ex_ref[...] = _masked_ex(
        _rbf(dd), (wa1_ref, ba1_ref, wa2_ref, ba2_ref, wa3_ref, ba3_ref))
    inv = 1.0 / (dd + 1e-9)
    x = diff[:, 0:1] * inv
    y = diff[:, 1:2] * inv
    z = diff[:, 2:3] * inv
    s3, s15, s5 = np.sqrt(3.0), np.sqrt(15.0), np.sqrt(5.0)
    one = jnp.ones_like(x)
    ess_ref[...] = jnp.concatenate([
        one, s3 * x, s3 * y, s3 * z,
        s15 * x * y, s15 * y * z,
        (s5 / 2.0) * (2.0 * z * z - x * x - y * y),
        s15 * x * z, (s15 / 2.0) * (x * x - y * y),
        dd, jnp.zeros_like(x), jnp.zeros_like(x), jnp.zeros_like(x),
        jnp.zeros_like(x), jnp.zeros_like(x), jnp.zeros_like(x),
    ], axis=1)


def _geom_call(ps, pd, attn_w):
    grid = _EP // _BE
    return pl.pallas_call(
        _geom_body,
        grid=(grid,),
        in_specs=[pl.BlockSpec((_BE, 16), lambda i: (i, 0)),
                  pl.BlockSpec((_BE, 16), lambda i: (i, 0))]
        + [_bcast(w.shape) for w in attn_w],
        out_specs=(pl.BlockSpec((_BE, 16), lambda i: (i, 0)),
                   pl.BlockSpec((_BE, 8), lambda i: (i, 0))),
        out_shape=(jax.ShapeDtypeStruct((_EP, 16), jnp.float32),
                   jax.ShapeDtypeStruct((_EP, 8), jnp.float32)),
        compiler_params=pltpu.CompilerParams(
            dimension_semantics=("arbitrary",)),
    )(ps, pd, *attn_w)


def _silu(x):
    return x * jax.nn.sigmoid(x)


def _lnorm(x):
    m = x.mean(-1, keepdims=True)
    v = ((x - m) ** 2).mean(-1, keepdims=True)
    return (x - m) * lax.rsqrt(v + 1e-6)


def _rbf(d):
    c = lax.broadcasted_iota(jnp.int32, (1, _L), 1).astype(jnp.float32) * (
        10.0 / (_L - 1))
    w = 0.5 * 10.0 / _L
    return jnp.exp(-((d - c) ** 2) / (2.0 * w * w))


def _dot(a, b):
    return jnp.dot(a, b, preferred_element_type=jnp.float32)


def _attn_tail(elen, wa1, ba1, wa2, ba2, wa3, ba3):
    a = _silu(_lnorm(_dot(elen, wa1) + ba1))
    a = _silu(_lnorm(_dot(a, wa2) + ba2))
    return _dot(a, wa3) + ba3


def _masked_ex(elen, attn_refs):
    wa1, ba1, wa2, ba2, wa3, ba3 = (r[...] for r in attn_refs)
    logits = _attn_tail(elen, wa1, ba1, wa2, ba2, wa3, ba3)
    lane = lax.broadcasted_iota(jnp.int32, logits.shape, 1)
    return jnp.where(lane < _H, jnp.exp(logits), 0.0)


def _edge_body(has_next, *args):
    (gs_ref, gd_ref, ess_ref,
     wps_ref, wpd_ref, wpe_ref, wm_ref, wsp_ref,
     wg1_ref, bg1_ref, wg2_ref, bg2_ref,
     wa1_ref, ba1_ref, wa2_ref, ba2_ref, wa3_ref, ba3_ref,
     sel_ref, expand_ref, wl_ref) = args[:21]
    rest = args[21:]
    ess = ess_ref[...]
    gd = gd_ref[...]
    elen = _rbf(ess[:, _SH:_SH + 1])
    b16 = lambda x: x.astype(jnp.bfloat16)

    logits = _attn_tail(elen, wa1_ref[...], ba1_ref[...], wa2_ref[...],
                        ba2_ref[...], wa3_ref[...], ba3_ref[...])
    alpha = jnp.exp(logits) * _dot(gd, sel_ref[...])

    msg = (_dot(gs_ref[...], wps_ref[...])
           + _dot(gd, wpd_ref[...])
           + _dot(b16(elen), wpe_ref[...]))
    gate = _dot(b16(_silu(_dot(elen, wg1_ref[...]) + bg1_ref[...])),
                wg2_ref[...]) + bg2_ref[...]
    val = _dot(b16(msg), wm_ref[...]) * _dot(b16(ess), wsp_ref[...]) * gate
    if has_next:
        nattn, eo_ref, exn_ref = rest[:6], rest[6], rest[7]
        exn_ref[...] = _masked_ex(elen, nattn)
    else:
        eo_ref = rest[0]
    out = _dot(b16(val * _dot(alpha, expand_ref[...])), wl_ref[...])
    eo_ref[...] = out.astype(jnp.bfloat16)


def _bcast(shape):
    nd = len(shape)
    return pl.BlockSpec(shape, lambda i: (0,) * nd)


def _edge_call(gs, gd, ess, weights, doutp, next_attn=None):
    grid = _EP // _BE
    has_next = next_attn is not None
    extra = tuple(next_attn) if has_next else ()
    out_specs = pl.BlockSpec((_BE, doutp), lambda i: (i, 0))
    out_shape = jax.ShapeDtypeStruct((_EP, doutp), jnp.bfloat16)
    if has_next:
        out_specs = (out_specs, pl.BlockSpec((_BE, 8), lambda i: (i, 0)))
        out_shape = (out_shape, jax.ShapeDtypeStruct((_EP, 8), jnp.float32))
    return pl.pallas_call(
        functools.partial(_edge_body, has_next),
        grid=(grid,),
        in_specs=[
            pl.BlockSpec((_BE, _DP), lambda i: (i, 0)),
            pl.BlockSpec((_BE, _DP), lambda i: (i, 0)),
            pl.BlockSpec((_BE, 16), lambda i: (i, 0)),
        ] + [_bcast(w.shape) for w in weights + extra],
        out_specs=out_specs,
        out_shape=out_shape,
        compiler_params=pltpu.CompilerParams(
            dimension_semantics=("arbitrary",)),
    )(gs, gd, ess, *weights, *extra)


def _pad2(a, rows, cols):
    return jnp.zeros((rows, cols), jnp.float32).at[:a.shape[0], :a.shape[1]].set(a)


def _layer_weights(b, dout):
    doutp = 128 if dout <= 128 else dout
    wpre = b['W_pre']
    wps = _pad2(wpre[:_D], _DP, _DP)
    wpd = _pad2(wpre[_D:2 * _D], _DP, _DP)
    wpe = _pad2(wpre[2 * _D:], _L, _DP)
    wm = _pad2(b['Wm'], _DP, _VP)
    wsp = _pad2(b['Ws'], 16, _VP)
    wg1 = b['Wg1']
    bg1 = b['bg1'][None, :]
    wg2 = _pad2(b['Wg2'], _L, _VP)
    bg2 = _pad2(b['bg2'][None, :], 1, _VP)
    wa1 = b['Wa1']
    ba1 = b['ba1'][None, :]
    wa2 = b['Wa2']
    ba2 = b['ba2'][None, :]
    wa3 = _pad2(b['Wa3'], _L, 8)
    ba3 = _pad2(b['ba3'][None, :], 1, 8)
    expand = np.zeros((8, _VP), np.float32)
    for h in range(_H):
        expand[h, h * _D:(h + 1) * _D] = 1.0
    expand = jnp.asarray(expand)
    sel = np.zeros((_DP, 8), np.float32)
    for h in range(_H):
        sel[_D + h, h] = 1.0
    sel = jnp.asarray(sel)
    wl = _pad2(b['W_lin'][:_H * _D], _VP, doutp)
    wl_bot = _pad2(b['W_lin'][_H * _D:], _DP, doutp)
    attn_w = (wa1, ba1, wa2, ba2, wa3, ba3)
    bf = lambda x: x.astype(jnp.bfloat16)
    edge_w = (bf(wps), bf(wpd), bf(wpe), bf(wm), bf(wsp), wg1, bg1,
              bf(wg2), bg2, wa1, ba1, wa2, ba2, wa3, ba3, bf(sel), expand,
              bf(wl))
    return attn_w, edge_w, wl_bot, doutp


def kernel(pos, params, node_atom, edge_src, edge_dst, batch):
    p = params
    node = _dot(p['emb_table'][node_atom], p['W_to_irreps'])

    src = edge_src.astype(jnp.int32)
    dst = edge_dst.astype(jnp.int32)
    pad = _EP - _E
    src_p = jnp.concatenate([src, jnp.zeros((pad,), jnp.int32)])
    dst_p = jnp.concatenate([dst, jnp.full((pad,), _N, jnp.int32)])

    pos_t = jnp.zeros((_N + 1, 16), jnp.float32).at[:_N, :3].set(pos)
    ps, pd = _sc_gather(pos_t, src_p, dst_p, width=16, tc_tiling=False)

    attn_w1, edge_w1, wl_bot1, doutp1 = _layer_weights(p['blocks'][0], _D)
    attn_w2, edge_w2, wl_bot2, doutp2 = _layer_weights(p['blocks'][1], _F)

    ess, ex1 = _geom_call(ps, pd, attn_w1)

    nodep = jnp.zeros((_N + 1, _DP), jnp.float32).at[:_N, :_D].set(node)

    dn = jax.ops.segment_sum(ex1, dst_p, num_segments=_N + 1)
    inv = 1.0 / (dn[:_N, :_H] + 1e-12)
    node_t = nodep.at[:_N, _D:_D + _H].set(inv)
    gs, gd = _sc_gather(node_t.astype(jnp.bfloat16), src_p, dst_p,
                        tc_tiling=False, dtype=jnp.bfloat16)
    eo, ex2 = _edge_call(gs, gd, ess, edge_w1, doutp1, next_attn=attn_w2)
    contrib = jax.ops.segment_sum(eo, dst_p, num_segments=_N + 1)
    nodep = contrib.astype(jnp.float32) + _dot(node_t, wl_bot1)

    dn = jax.ops.segment_sum(ex2, dst_p, num_segments=_N + 1)
    inv = 1.0 / (dn[:_N, :_H] + 1e-12)
    node_t = nodep.at[:_N, _D:_D + _H].set(inv)
    gs, gd = _sc_gather(node_t.astype(jnp.bfloat16), src_p, dst_p,
                        tc_tiling=False, dtype=jnp.bfloat16)
    eo = _edge_call(gs, gd, ess, edge_w2, doutp2)
    contrib = jax.ops.segment_sum(eo, dst_p, num_segments=_N + 1)
    node2 = contrib.astype(jnp.float32) + _dot(node_t, wl_bot2)

    h = _silu(_dot(node2[:_N], p['Wh1']) + p['bh1'])
    h = _silu(_dot(h, p['Wh2']) + p['bh2'])
    ne = _dot(h, p['Wh3']) + p['bh3']
    return jax.ops.segment_sum(ne, batch, num_segments=_G)


# speedup vs baseline: 1.5573x; 1.5573x over previous
import functools

import jax
import jax.numpy as jnp
import numpy as np
from jax import lax
from jax.experimental import pallas as pl
from jax.experimental.pallas import tpu as pltpu
from jax.experimental.pallas import tpu_sc as plsc

_N = 10000
_E = 160000
_G = 16
_D = 120
_H = 4
_L = 64
_SH = 9
_F = 256

_BE = 1024
_EP = 160 * _BE
_DP = 128
_VP = 512


_NC, _NS = 2, 16
_NW = _NC * _NS
_EW = _EP // _NW
_CG = 128


def _gather_body(node_hbm, src_hbm, dst_hbm,
                 gs_hbm, gd_hbm,
                 idx_s0, idx_d0, idx_s1, idx_d1,
                 rows_s, rows_d, sem_g0, sem_g1):
    wid = lax.axis_index("s") * _NC + lax.axis_index("c")
    base = wid * _EW
    nchunk = _EW // _CG
    sems = (sem_g0, sem_g1)
    idx_s = (idx_s0, idx_s1)
    idx_d = (idx_d0, idx_d1)

    def start(c, b):
        off = base + c * _CG
        pltpu.sync_copy(src_hbm.at[pl.ds(off, _CG)], idx_s[b])
        pltpu.sync_copy(dst_hbm.at[pl.ds(off, _CG)], idx_d[b])
        pltpu.async_copy(node_hbm.at[idx_s[b]], rows_s.at[b], sems[b])
        pltpu.async_copy(node_hbm.at[idx_d[b]], rows_d.at[b], sems[b])

    def finish(c, b):
        off = base + c * _CG
        pltpu.make_async_copy(node_hbm.at[idx_s[b]], rows_s.at[b],
                              sems[b]).wait()
        pltpu.make_async_copy(node_hbm.at[idx_d[b]], rows_d.at[b],
                              sems[b]).wait()
        pltpu.sync_copy(rows_s.at[b], gs_hbm.at[pl.ds(off, _CG)])
        pltpu.sync_copy(rows_d.at[b], gd_hbm.at[pl.ds(off, _CG)])

    start(0, 0)

    def step(j, carry):
        c0, c1 = 2 * j, 2 * j + 1
        start(c1, 1)
        finish(c0, 0)

        @pl.when(j < nchunk // 2 - 1)
        def _():
            start(c0 + 2, 0)

        finish(c1, 1)
        return carry

    lax.fori_loop(0, nchunk // 2, step, 0)


def _sc_gather(node_t, src_p, dst_p, width=_DP, tc_tiling=True,
               dtype=jnp.float32):
    f = pl.kernel(
        _gather_body,
        out_type=(jax.ShapeDtypeStruct((_EP, width), dtype),
                  jax.ShapeDtypeStruct((_EP, width), dtype)),
        mesh=plsc.VectorSubcoreMesh(core_axis_name="c", subcore_axis_name="s"),
        scratch_types=[
            pltpu.VMEM((_CG,), jnp.int32),
            pltpu.VMEM((_CG,), jnp.int32),
            pltpu.VMEM((_CG,), jnp.int32),
            pltpu.VMEM((_CG,), jnp.int32),
            pltpu.VMEM((2, _CG, width), dtype),
            pltpu.VMEM((2, _CG, width), dtype),
            pltpu.SemaphoreType.DMA,
            pltpu.SemaphoreType.DMA,
        ],
        compiler_params=pltpu.CompilerParams(use_tc_tiling_on_sc=tc_tiling),
    )
    return f(node_t, src_p, dst_p)


def _geom_body(ps_ref, pd_ref, wa1_ref, ba1_ref, wa2_ref, ba2_ref,
               wa3_ref, ba3_ref, ess_ref, ex_ref):
    diff = pd_ref[...] - ps_ref[...]
    d2 = jnp.sum(diff * diff, axis=-1, keepdims=True)
    dd = jnp.sqrt(d2)
    ex_ref[...] = _masked_ex(
        _rbf(dd), (wa1_ref, ba1_ref, wa2_ref, ba2_ref, wa3_ref, ba3_ref))
    inv = 1.0 / (dd + 1e-9)
    x = diff[:, 0:1] * inv
    y = diff[:, 1:2] * inv
    z = diff[:, 2:3] * inv
    s3, s15, s5 = np.sqrt(3.0), np.sqrt(15.0), np.sqrt(5.0)
    one = jnp.ones_like(x)
    ess_ref[...] = jnp.concatenate([
        one, s3 * x, s3 * y, s3 * z,
        s15 * x * y, s15 * y * z,
        (s5 / 2.0) * (2.0 * z * z - x * x - y * y),
        s15 * x * z, (s15 / 2.0) * (x * x - y * y),
        dd, jnp.zeros_like(x), jnp.zeros_like(x), jnp.zeros_like(x),
        jnp.zeros_like(x), jnp.zeros_like(x), jnp.zeros_like(x),
    ], axis=1)


def _geom_call(ps, pd, attn_w):
    grid = _EP // _BE
    return pl.pallas_call(
        _geom_body,
        grid=(grid,),
        in_specs=[pl.BlockSpec((_BE, 16), lambda i: (i, 0)),
                  pl.BlockSpec((_BE, 16), lambda i: (i, 0))]
        + [_bcast(w.shape) for w in attn_w],
        out_specs=(pl.BlockSpec((_BE, 16), lambda i: (i, 0)),
                   pl.BlockSpec((_BE, 8), lambda i: (i, 0))),
        out_shape=(jax.ShapeDtypeStruct((_EP, 16), jnp.float32),
                   jax.ShapeDtypeStruct((_EP, 8), jnp.float32)),
        compiler_params=pltpu.CompilerParams(
            dimension_semantics=("arbitrary",)),
    )(ps, pd, *attn_w)


def _silu(x):
    return x * jax.nn.sigmoid(x)


def _lnorm(x):
    m = x.mean(-1, keepdims=True)
    v = ((x - m) ** 2).mean(-1, keepdims=True)
    return (x - m) * lax.rsqrt(v + 1e-6)


def _rbf(d):
    c = lax.broadcasted_iota(jnp.int32, (1, _L), 1).astype(jnp.float32) * (
        10.0 / (_L - 1))
    w = 0.5 * 10.0 / _L
    return jnp.exp(-((d - c) ** 2) / (2.0 * w * w))


def _dot(a, b):
    return jnp.dot(a, b, preferred_element_type=jnp.float32)


def _attn_tail(elen, wa1, ba1, wa2, ba2, wa3, ba3):
    a = _silu(_lnorm(_dot(elen, wa1) + ba1))
    a = _silu(_lnorm(_dot(a, wa2) + ba2))
    return _dot(a, wa3) + ba3


def _masked_ex(elen, attn_refs):
    wa1, ba1, wa2, ba2, wa3, ba3 = (r[...] for r in attn_refs)
    logits = _attn_tail(elen, wa1, ba1, wa2, ba2, wa3, ba3)
    lane = lax.broadcasted_iota(jnp.int32, logits.shape, 1)
    return jnp.where(lane < _H, jnp.exp(logits), 0.0)


def _edge_body(has_next, *args):
    (gs_ref, gd_ref, ess_ref,
     wps_ref, wpd_ref, wpe_ref, wm_ref, wsp_ref,
     wg1_ref, bg1_ref, wg2_ref, bg2_ref,
     wa1_ref, ba1_ref, wa2_ref, ba2_ref, wa3_ref, ba3_ref,
     sel_ref, expand_ref, wl_ref) = args[:21]
    rest = args[21:]
    ess = ess_ref[...]
    gd = gd_ref[...]
    elen = _rbf(ess[:, _SH:_SH + 1])
    b16 = lambda x: x.astype(jnp.bfloat16)

    logits = _attn_tail(elen, wa1_ref[...], ba1_ref[...], wa2_ref[...],
                        ba2_ref[...], wa3_ref[...], ba3_ref[...])
    alpha = jnp.exp(logits) * _dot(gd, sel_ref[...])

    msg = (_dot(gs_ref[...], wps_ref[...])
           + _dot(gd, wpd_ref[...])
           + _dot(b16(elen), wpe_ref[...]))
    gate = _dot(b16(_silu(_dot(elen, wg1_ref[...]) + bg1_ref[...])),
                wg2_ref[...]) + bg2_ref[...]
    val = _dot(b16(msg), wm_ref[...]) * _dot(b16(ess), wsp_ref[...]) * gate
    if has_next:
        nattn, eo_ref, exn_ref = rest[:6], rest[6], rest[7]
        exn_ref[...] = _masked_ex(elen, nattn)
    else:
        eo_ref = rest[0]
    eo_ref[...] = _dot(b16(val * _dot(alpha, expand_ref[...])), wl_ref[...])


def _bcast(shape):
    nd = len(shape)
    return pl.BlockSpec(shape, lambda i: (0,) * nd)


def _edge_call(gs, gd, ess, weights, doutp, next_attn=None):
    grid = _EP // _BE
    has_next = next_attn is not None
    extra = tuple(next_attn) if has_next else ()
    out_specs = pl.BlockSpec((_BE, doutp), lambda i: (i, 0))
    out_shape = jax.ShapeDtypeStruct((_EP, doutp), jnp.float32)
    if has_next:
        out_specs = (out_specs, pl.BlockSpec((_BE, 8), lambda i: (i, 0)))
        out_shape = (out_shape, jax.ShapeDtypeStruct((_EP, 8), jnp.float32))
    return pl.pallas_call(
        functools.partial(_edge_body, has_next),
        grid=(grid,),
        in_specs=[
            pl.BlockSpec((_BE, _DP), lambda i: (i, 0)),
            pl.BlockSpec((_BE, _DP), lambda i: (i, 0)),
            pl.BlockSpec((_BE, 16), lambda i: (i, 0)),
        ] + [_bcast(w.shape) for w in weights + extra],
        out_specs=out_specs,
        out_shape=out_shape,
        compiler_params=pltpu.CompilerParams(
            dimension_semantics=("arbitrary",)),
    )(gs, gd, ess, *weights, *extra)


def _pad2(a, rows, cols):
    return jnp.zeros((rows, cols), jnp.float32).at[:a.shape[0], :a.shape[1]].set(a)


def _layer_weights(b, dout):
    doutp = 128 if dout <= 128 else dout
    wpre = b['W_pre']
    wps = _pad2(wpre[:_D], _DP, _DP)
    wpd = _pad2(wpre[_D:2 * _D], _DP, _DP)
    wpe = _pad2(wpre[2 * _D:], _L, _DP)
    wm = _pad2(b['Wm'], _DP, _VP)
    wsp = _pad2(b['Ws'], 16, _VP)
    wg1 = b['Wg1']
    bg1 = b['bg1'][None, :]
    wg2 = _pad2(b['Wg2'], _L, _VP)
    bg2 = _pad2(b['bg2'][None, :], 1, _VP)
    wa1 = b['Wa1']
    ba1 = b['ba1'][None, :]
    wa2 = b['Wa2']
    ba2 = b['ba2'][None, :]
    wa3 = _pad2(b['Wa3'], _L, 8)
    ba3 = _pad2(b['ba3'][None, :], 1, 8)
    expand = np.zeros((8, _VP), np.float32)
    for h in range(_H):
        expand[h, h * _D:(h + 1) * _D] = 1.0
    expand = jnp.asarray(expand)
    sel = np.zeros((_DP, 8), np.float32)
    for h in range(_H):
        sel[_D + h, h] = 1.0
    sel = jnp.asarray(sel)
    wl = _pad2(b['W_lin'][:_H * _D], _VP, doutp)
    wl_bot = _pad2(b['W_lin'][_H * _D:], _DP, doutp)
    attn_w = (wa1, ba1, wa2, ba2, wa3, ba3)
    bf = lambda x: x.astype(jnp.bfloat16)
    edge_w = (bf(wps), bf(wpd), bf(wpe), bf(wm), bf(wsp), wg1, bg1,
              bf(wg2), bg2, wa1, ba1, wa2, ba2, wa3, ba3, bf(sel), expand,
              bf(wl))
    return attn_w, edge_w, wl_bot, doutp


def kernel(pos, params, node_atom, edge_src, edge_dst, batch):
    p = params
    node = _dot(p['emb_table'][node_atom], p['W_to_irreps'])

    src = edge_src.astype(jnp.int32)
    dst = edge_dst.astype(jnp.int32)
    pad = _EP - _E
    src_p = jnp.concatenate([src, jnp.zeros((pad,), jnp.int32)])
    dst_p = jnp.concatenate([dst, jnp.full((pad,), _N, jnp.int32)])

    pos_t = jnp.zeros((_N + 1, 16), jnp.float32).at[:_N, :3].set(pos)
    ps, pd = _sc_gather(pos_t, src_p, dst_p, width=16, tc_tiling=False)

    attn_w1, edge_w1, wl_bot1, doutp1 = _layer_weights(p['blocks'][0], _D)
    attn_w2, edge_w2, wl_bot2, doutp2 = _layer_weights(p['blocks'][1], _F)

    ess, ex1 = _geom_call(ps, pd, attn_w1)

    nodep = jnp.zeros((_N + 1, _DP), jnp.float32).at[:_N, :_D].set(node)

    dn = jax.ops.segment_sum(ex1, dst_p, num_segments=_N + 1)
    inv = 1.0 / (dn[:_N, :_H] + 1e-12)
    node_t = nodep.at[:_N, _D:_D + _H].set(inv)
    gs, gd = _sc_gather(node_t.astype(jnp.bfloat16), src_p, dst_p,
                        tc_tiling=False, dtype=jnp.bfloat16)
    eo, ex2 = _edge_call(gs, gd, ess, edge_w1, doutp1, next_attn=attn_w2)
    contrib = jax.ops.segment_sum(eo, dst_p, num_segments=_N + 1)
    nodep = contrib + _dot(node_t, wl_bot1)

    dn = jax.ops.segment_sum(ex2, dst_p, num_segments=_N + 1)
    inv = 1.0 / (dn[:_N, :_H] + 1e-12)
    node_t = nodep.at[:_N, _D:_D + _H].set(inv)
    gs, gd = _sc_gather(node_t.astype(jnp.bfloat16), src_p, dst_p,
                        tc_tiling=False, dtype=jnp.bfloat16)
    eo = _edge_call(gs, gd, ess, edge_w2, doutp2)
    contrib = jax.ops.segment_sum(eo, dst_p, num_segments=_N + 1)
    node2 = contrib + _dot(node_t, wl_bot2)

    h = _silu(_dot(node2[:_N], p['Wh1']) + p['bh1'])
    h = _silu(_dot(h, p['Wh2']) + p['bh2'])
    ne = _dot(h, p['Wh3']) + p['bh3']
    return jax.ops.segment_sum(ne, batch, num_segments=_G)


# speedup vs baseline: 1.7778x; 1.1415x over previous
import functools

import jax
import jax.numpy as jnp
import numpy as np
from jax import lax
from jax.experimental import pallas as pl
from jax.experimental.pallas import tpu as pltpu
from jax.experimental.pallas import tpu_sc as plsc

_N = 10000
_E = 160000
_G = 16
_D = 120
_H = 4
_L = 64
_SH = 9
_F = 256

_BE = 2048
_EP = 80 * _BE
_DP = 128
_VP = 512


_NC, _NS = 2, 16
_NW = _NC * _NS
_EW = _EP // _NW
_CG = 128


def _gather_body(node_hbm, src_hbm, dst_hbm,
                 gs_hbm, gd_hbm,
                 idx_s0, idx_d0, idx_s1, idx_d1,
                 rows_s, rows_d, sem_g0, sem_g1):
    wid = lax.axis_index("s") * _NC + lax.axis_index("c")
    base = wid * _EW
    nchunk = _EW // _CG
    sems = (sem_g0, sem_g1)
    idx_s = (idx_s0, idx_s1)
    idx_d = (idx_d0, idx_d1)

    def start(c, b):
        off = base + c * _CG
        pltpu.sync_copy(src_hbm.at[pl.ds(off, _CG)], idx_s[b])
        pltpu.sync_copy(dst_hbm.at[pl.ds(off, _CG)], idx_d[b])
        pltpu.async_copy(node_hbm.at[idx_s[b]], rows_s.at[b], sems[b])
        pltpu.async_copy(node_hbm.at[idx_d[b]], rows_d.at[b], sems[b])

    def finish(c, b):
        off = base + c * _CG
        pltpu.make_async_copy(node_hbm.at[idx_s[b]], rows_s.at[b],
                              sems[b]).wait()
        pltpu.make_async_copy(node_hbm.at[idx_d[b]], rows_d.at[b],
                              sems[b]).wait()
        pltpu.sync_copy(rows_s.at[b], gs_hbm.at[pl.ds(off, _CG)])
        pltpu.sync_copy(rows_d.at[b], gd_hbm.at[pl.ds(off, _CG)])

    start(0, 0)

    def step(j, carry):
        c0, c1 = 2 * j, 2 * j + 1
        start(c1, 1)
        finish(c0, 0)

        @pl.when(j < nchunk // 2 - 1)
        def _():
            start(c0 + 2, 0)

        finish(c1, 1)
        return carry

    lax.fori_loop(0, nchunk // 2, step, 0)


def _sc_gather(node_t, src_p, dst_p, width=_DP, tc_tiling=True,
               dtype=jnp.float32):
    f = pl.kernel(
        _gather_body,
        out_type=(jax.ShapeDtypeStruct((_EP, width), dtype),
                  jax.ShapeDtypeStruct((_EP, width), dtype)),
        mesh=plsc.VectorSubcoreMesh(core_axis_name="c", subcore_axis_name="s"),
        scratch_types=[
            pltpu.VMEM((_CG,), jnp.int32),
            pltpu.VMEM((_CG,), jnp.int32),
            pltpu.VMEM((_CG,), jnp.int32),
            pltpu.VMEM((_CG,), jnp.int32),
            pltpu.VMEM((2, _CG, width), dtype),
            pltpu.VMEM((2, _CG, width), dtype),
            pltpu.SemaphoreType.DMA,
            pltpu.SemaphoreType.DMA,
        ],
        compiler_params=pltpu.CompilerParams(use_tc_tiling_on_sc=tc_tiling),
    )
    return f(node_t, src_p, dst_p)


def _geom_body(ps_ref, pd_ref, wa1_ref, ba1_ref, wa2_ref, ba2_ref,
               wa3_ref, ba3_ref, ess_ref, ex_ref):
    diff = pd_ref[...] - ps_ref[...]
    d2 = jnp.sum(diff * diff, axis=-1, keepdims=True)
    dd = jnp.sqrt(d2)
    ex_ref[...] = _masked_ex(
        _rbf(dd), (wa1_ref, ba1_ref, wa2_ref, ba2_ref, wa3_ref, ba3_ref))
    inv = 1.0 / (dd + 1e-9)
    x = diff[:, 0:1] * inv
    y = diff[:, 1:2] * inv
    z = diff[:, 2:3] * inv
    s3, s15, s5 = np.sqrt(3.0), np.sqrt(15.0), np.sqrt(5.0)
    one = jnp.ones_like(x)
    ess_ref[...] = jnp.concatenate([
        one, s3 * x, s3 * y, s3 * z,
        s15 * x * y, s15 * y * z,
        (s5 / 2.0) * (2.0 * z * z - x * x - y * y),
        s15 * x * z, (s15 / 2.0) * (x * x - y * y),
        dd, jnp.zeros_like(x), jnp.zeros_like(x), jnp.zeros_like(x),
        jnp.zeros_like(x), jnp.zeros_like(x), jnp.zeros_like(x),
    ], axis=1)


def _geom_call(ps, pd, attn_w):
    grid = _EP // _BE
    return pl.pallas_call(
        _geom_body,
        grid=(grid,),
        in_specs=[pl.BlockSpec((_BE, 16), lambda i: (i, 0)),
                  pl.BlockSpec((_BE, 16), lambda i: (i, 0))]
        + [_bcast(w.shape) for w in attn_w],
        out_specs=(pl.BlockSpec((_BE, 16), lambda i: (i, 0)),
                   pl.BlockSpec((_BE, 8), lambda i: (i, 0))),
        out_shape=(jax.ShapeDtypeStruct((_EP, 16), jnp.float32),
                   jax.ShapeDtypeStruct((_EP, 8), jnp.float32)),
        compiler_params=pltpu.CompilerParams(
            dimension_semantics=("arbitrary",)),
    )(ps, pd, *attn_w)


def _silu(x):
    return x * jax.nn.sigmoid(x)


def _lnorm(x):
    m = x.mean(-1, keepdims=True)
    v = ((x - m) ** 2).mean(-1, keepdims=True)
    return (x - m) * lax.rsqrt(v + 1e-6)


def _rbf(d):
    c = lax.broadcasted_iota(jnp.int32, (1, _L), 1).astype(jnp.float32) * (
        10.0 / (_L - 1))
    w = 0.5 * 10.0 / _L
    return jnp.exp(-((d - c) ** 2) / (2.0 * w * w))


def _dot(a, b):
    return jnp.dot(a, b, preferred_element_type=jnp.float32)


def _attn_tail(elen, wa1, ba1, wa2, ba2, wa3, ba3):
    a = _silu(_lnorm(_dot(elen, wa1) + ba1))
    a = _silu(_lnorm(_dot(a, wa2) + ba2))
    return _dot(a, wa3) + ba3


def _masked_ex(elen, attn_refs):
    wa1, ba1, wa2, ba2, wa3, ba3 = (r[...] for r in attn_refs)
    logits = _attn_tail(elen, wa1, ba1, wa2, ba2, wa3, ba3)
    lane = lax.broadcasted_iota(jnp.int32, logits.shape, 1)
    return jnp.where(lane < _H, jnp.exp(logits), 0.0)


def _edge_body(has_next, *args):
    (gs_ref, gd_ref, ess_ref,
     wps_ref, wpd_ref, wpe_ref, wm_ref, wsp_ref,
     wg1_ref, bg1_ref, wg2_ref, bg2_ref,
     wa1_ref, ba1_ref, wa2_ref, ba2_ref, wa3_ref, ba3_ref,
     sel_ref, expand_ref, wl_ref) = args[:21]
    rest = args[21:]
    ess = ess_ref[...]
    gd = gd_ref[...]
    elen = _rbf(ess[:, _SH:_SH + 1])

    logits = _attn_tail(elen, wa1_ref[...], ba1_ref[...], wa2_ref[...],
                        ba2_ref[...], wa3_ref[...], ba3_ref[...])
    alpha = jnp.exp(logits) * _dot(gd, sel_ref[...])

    msg = (_dot(gs_ref[...], wps_ref[...])
           + _dot(gd, wpd_ref[...])
           + _dot(elen, wpe_ref[...]))
    gate = _dot(_silu(_dot(elen, wg1_ref[...]) + bg1_ref[...]),
                wg2_ref[...]) + bg2_ref[...]
    val = _dot(msg, wm_ref[...]) * _dot(ess, wsp_ref[...]) * gate
    if has_next:
        nattn, eo_ref, exn_ref = rest[:6], rest[6], rest[7]
        exn_ref[...] = _masked_ex(elen, nattn)
    else:
        eo_ref = rest[0]
    eo_ref[...] = _dot(val * _dot(alpha, expand_ref[...]), wl_ref[...])


def _bcast(shape):
    nd = len(shape)
    return pl.BlockSpec(shape, lambda i: (0,) * nd)


def _edge_call(gs, gd, ess, weights, doutp, next_attn=None):
    grid = _EP // _BE
    has_next = next_attn is not None
    extra = tuple(next_attn) if has_next else ()
    out_specs = pl.BlockSpec((_BE, doutp), lambda i: (i, 0))
    out_shape = jax.ShapeDtypeStruct((_EP, doutp), jnp.float32)
    if has_next:
        out_specs = (out_specs, pl.BlockSpec((_BE, 8), lambda i: (i, 0)))
        out_shape = (out_shape, jax.ShapeDtypeStruct((_EP, 8), jnp.float32))
    return pl.pallas_call(
        functools.partial(_edge_body, has_next),
        grid=(grid,),
        in_specs=[
            pl.BlockSpec((_BE, _DP), lambda i: (i, 0)),
            pl.BlockSpec((_BE, _DP), lambda i: (i, 0)),
            pl.BlockSpec((_BE, 16), lambda i: (i, 0)),
        ] + [_bcast(w.shape) for w in weights + extra],
        out_specs=out_specs,
        out_shape=out_shape,
        compiler_params=pltpu.CompilerParams(
            dimension_semantics=("arbitrary",)),
    )(gs, gd, ess, *weights, *extra)


def _pad2(a, rows, cols):
    return jnp.zeros((rows, cols), jnp.float32).at[:a.shape[0], :a.shape[1]].set(a)


def _layer_weights(b, dout):
    doutp = 128 if dout <= 128 else dout
    wpre = b['W_pre']
    wps = _pad2(wpre[:_D], _DP, _DP)
    wpd = _pad2(wpre[_D:2 * _D], _DP, _DP)
    wpe = _pad2(wpre[2 * _D:], _L, _DP)
    wm = _pad2(b['Wm'], _DP, _VP)
    wsp = _pad2(b['Ws'], 16, _VP)
    wg1 = b['Wg1']
    bg1 = b['bg1'][None, :]
    wg2 = _pad2(b['Wg2'], _L, _VP)
    bg2 = _pad2(b['bg2'][None, :], 1, _VP)
    wa1 = b['Wa1']
    ba1 = b['ba1'][None, :]
    wa2 = b['Wa2']
    ba2 = b['ba2'][None, :]
    wa3 = _pad2(b['Wa3'], _L, 8)
    ba3 = _pad2(b['ba3'][None, :], 1, 8)
    expand = np.zeros((8, _VP), np.float32)
    for h in range(_H):
        expand[h, h * _D:(h + 1) * _D] = 1.0
    expand = jnp.asarray(expand)
    sel = np.zeros((_DP, 8), np.float32)
    for h in range(_H):
        sel[_D + h, h] = 1.0
    sel = jnp.asarray(sel)
    wl = _pad2(b['W_lin'][:_H * _D], _VP, doutp)
    wl_bot = _pad2(b['W_lin'][_H * _D:], _DP, doutp)
    attn_w = (wa1, ba1, wa2, ba2, wa3, ba3)
    edge_w = (wps, wpd, wpe, wm, wsp, wg1, bg1, wg2, bg2,
              wa1, ba1, wa2, ba2, wa3, ba3, sel, expand, wl)
    return attn_w, edge_w, wl_bot, doutp


def kernel(pos, params, node_atom, edge_src, edge_dst, batch):
    p = params
    node = _dot(p['emb_table'][node_atom], p['W_to_irreps'])

    src = edge_src.astype(jnp.int32)
    dst = edge_dst.astype(jnp.int32)
    pad = _EP - _E
    src_p = jnp.concatenate([src, jnp.zeros((pad,), jnp.int32)])
    dst_p = jnp.concatenate([dst, jnp.full((pad,), _N, jnp.int32)])

    pos_t = jnp.zeros((_N + 1, 16), jnp.float32).at[:_N, :3].set(pos)
    ps, pd = _sc_gather(pos_t, src_p, dst_p, width=16, tc_tiling=False)

    attn_w1, edge_w1, wl_bot1, doutp1 = _layer_weights(p['blocks'][0], _D)
    attn_w2, edge_w2, wl_bot2, doutp2 = _layer_weights(p['blocks'][1], _F)

    ess, ex1 = _geom_call(ps, pd, attn_w1)

    nodep = jnp.zeros((_N + 1, _DP), jnp.float32).at[:_N, :_D].set(node)

    dn = jax.ops.segment_sum(ex1, dst_p, num_segments=_N + 1)
    inv = 1.0 / (dn[:_N, :_H] + 1e-12)
    node_t = nodep.at[:_N, _D:_D + _H].set(inv)
    gs, gd = _sc_gather(node_t, src_p, dst_p)
    eo, ex2 = _edge_call(gs, gd, ess, edge_w1, doutp1, next_attn=attn_w2)
    contrib = jax.ops.segment_sum(eo, dst_p, num_segments=_N + 1)
    nodep = contrib + _dot(node_t, wl_bot1)

    dn = jax.ops.segment_sum(ex2, dst_p, num_segments=_N + 1)
    inv = 1.0 / (dn[:_N, :_H] + 1e-12)
    node_t = nodep.at[:_N, _D:_D + _H].set(inv)
    gs, gd = _sc_gather(node_t, src_p, dst_p)
    eo = _edge_call(gs, gd, ess, edge_w2, doutp2)
    contrib = jax.ops.segment_sum(eo, dst_p, num_segments=_N + 1)
    node2 = contrib + _dot(node_t, wl_bot2)

    h = _silu(_dot(node2[:_N], p['Wh1']) + p['bh1'])
    h = _silu(_dot(h, p['Wh2']) + p['bh2'])
    ne = _dot(h, p['Wh3']) + p['bh3']
    return jax.ops.segment_sum(ne, batch, num_segments=_G)


# speedup vs baseline: 1.8009x; 1.0130x over previous
import functools

import jax
import jax.numpy as jnp
import numpy as np
from jax import lax
from jax.experimental import pallas as pl
from jax.experimental.pallas import tpu as pltpu
from jax.experimental.pallas import tpu_sc as plsc

_N = 10000
_E = 160000
_G = 16
_D = 120
_H = 4
_L = 64
_SH = 9
_F = 256

_BE = 2048
_EP = 80 * _BE
_DP = 128
_VP = 512


_NC, _NS = 2, 16
_NW = _NC * _NS
_EW = _EP // _NW
_CG = 128


def _gather_body(node_hbm, src_hbm, dst_hbm,
                 gs_hbm, gd_hbm,
                 idx_s0, idx_d0, idx_s1, idx_d1,
                 rows_s, rows_d, sem_g0, sem_g1):
    wid = lax.axis_index("s") * _NC + lax.axis_index("c")
    base = wid * _EW
    nchunk = _EW // _CG
    sems = (sem_g0, sem_g1)
    idx_s = (idx_s0, idx_s1)
    idx_d = (idx_d0, idx_d1)

    def start(c, b):
        off = base + c * _CG
        pltpu.sync_copy(src_hbm.at[pl.ds(off, _CG)], idx_s[b])
        pltpu.sync_copy(dst_hbm.at[pl.ds(off, _CG)], idx_d[b])
        pltpu.async_copy(node_hbm.at[idx_s[b]], rows_s.at[b], sems[b])
        pltpu.async_copy(node_hbm.at[idx_d[b]], rows_d.at[b], sems[b])

    def finish(c, b):
        off = base + c * _CG
        pltpu.make_async_copy(node_hbm.at[idx_s[b]], rows_s.at[b],
                              sems[b]).wait()
        pltpu.make_async_copy(node_hbm.at[idx_d[b]], rows_d.at[b],
                              sems[b]).wait()
        pltpu.sync_copy(rows_s.at[b], gs_hbm.at[pl.ds(off, _CG)])
        pltpu.sync_copy(rows_d.at[b], gd_hbm.at[pl.ds(off, _CG)])

    start(0, 0)

    def step(j, carry):
        c0, c1 = 2 * j, 2 * j + 1
        start(c1, 1)
        finish(c0, 0)

        @pl.when(j < nchunk // 2 - 1)
        def _():
            start(c0 + 2, 0)

        finish(c1, 1)
        return carry

    lax.fori_loop(0, nchunk // 2, step, 0)


def _sc_gather(node_t, src_p, dst_p, width=_DP, tc_tiling=True,
               dtype=jnp.float32):
    f = pl.kernel(
        _gather_body,
        out_type=(jax.ShapeDtypeStruct((_EP, width), dtype),
                  jax.ShapeDtypeStruct((_EP, width), dtype)),
        mesh=plsc.VectorSubcoreMesh(core_axis_name="c", subcore_axis_name="s"),
        scratch_types=[
            pltpu.VMEM((_CG,), jnp.int32),
            pltpu.VMEM((_CG,), jnp.int32),
            pltpu.VMEM((_CG,), jnp.int32),
            pltpu.VMEM((_CG,), jnp.int32),
            pltpu.VMEM((2, _CG, width), dtype),
            pltpu.VMEM((2, _CG, width), dtype),
            pltpu.SemaphoreType.DMA,
            pltpu.SemaphoreType.DMA,
        ],
        compiler_params=pltpu.CompilerParams(use_tc_tiling_on_sc=tc_tiling),
    )
    return f(node_t, src_p, dst_p)


def _gather1_body(tab_hbm, idx_hbm, out_hbm,
                  i0, i1, rows, sem0, sem1):
    wid = lax.axis_index("s") * _NC + lax.axis_index("c")
    base = wid * _EW
    nchunk = _EW // _CG
    sems = (sem0, sem1)
    idxs = (i0, i1)

    def start(c, b):
        off = base + c * _CG
        pltpu.sync_copy(idx_hbm.at[pl.ds(off, _CG)], idxs[b])
        pltpu.async_copy(tab_hbm.at[idxs[b]], rows.at[b], sems[b])

    def finish(c, b):
        off = base + c * _CG
        pltpu.make_async_copy(tab_hbm.at[idxs[b]], rows.at[b],
                              sems[b]).wait()
        pltpu.sync_copy(rows.at[b], out_hbm.at[pl.ds(off, _CG)])

    start(0, 0)

    def step(j, carry):
        c0, c1 = 2 * j, 2 * j + 1
        start(c1, 1)
        finish(c0, 0)

        @pl.when(j < nchunk // 2 - 1)
        def _():
            start(c0 + 2, 0)

        finish(c1, 1)
        return carry

    lax.fori_loop(0, nchunk // 2, step, 0)


def _sc_gather1(tab, idx, width=16):
    f = pl.kernel(
        _gather1_body,
        out_type=jax.ShapeDtypeStruct((_EP, width), jnp.float32),
        mesh=plsc.VectorSubcoreMesh(core_axis_name="c", subcore_axis_name="s"),
        scratch_types=[
            pltpu.VMEM((_CG,), jnp.int32),
            pltpu.VMEM((_CG,), jnp.int32),
            pltpu.VMEM((2, _CG, width), jnp.float32),
            pltpu.SemaphoreType.DMA,
            pltpu.SemaphoreType.DMA,
        ],
        compiler_params=pltpu.CompilerParams(use_tc_tiling_on_sc=False),
    )
    return f(tab, idx)


def _geom_body(ps_ref, pd_ref, wa1_ref, ba1_ref, wa2_ref, ba2_ref,
               wa3_ref, ba3_ref, ess_ref, ex_ref):
    diff = pd_ref[...] - ps_ref[...]
    d2 = jnp.sum(diff * diff, axis=-1, keepdims=True)
    dd = jnp.sqrt(d2)
    ex_ref[...] = _masked_ex(
        _rbf(dd), (wa1_ref, ba1_ref, wa2_ref, ba2_ref, wa3_ref, ba3_ref))
    inv = 1.0 / (dd + 1e-9)
    x = diff[:, 0:1] * inv
    y = diff[:, 1:2] * inv
    z = diff[:, 2:3] * inv
    s3, s15, s5 = np.sqrt(3.0), np.sqrt(15.0), np.sqrt(5.0)
    one = jnp.ones_like(x)
    ess_ref[...] = jnp.concatenate([
        one, s3 * x, s3 * y, s3 * z,
        s15 * x * y, s15 * y * z,
        (s5 / 2.0) * (2.0 * z * z - x * x - y * y),
        s15 * x * z, (s15 / 2.0) * (x * x - y * y),
        dd, jnp.zeros_like(x), jnp.zeros_like(x), jnp.zeros_like(x),
        jnp.zeros_like(x), jnp.zeros_like(x), jnp.zeros_like(x),
    ], axis=1)


def _geom_call(ps, pd, attn_w):
    grid = _EP // _BE
    return pl.pallas_call(
        _geom_body,
        grid=(grid,),
        in_specs=[pl.BlockSpec((_BE, 16), lambda i: (i, 0)),
                  pl.BlockSpec((_BE, 16), lambda i: (i, 0))]
        + [_bcast(w.shape) for w in attn_w],
        out_specs=(pl.BlockSpec((_BE, 16), lambda i: (i, 0)),
                   pl.BlockSpec((_BE, 8), lambda i: (i, 0))),
        out_shape=(jax.ShapeDtypeStruct((_EP, 16), jnp.float32),
                   jax.ShapeDtypeStruct((_EP, 8), jnp.float32)),
        compiler_params=pltpu.CompilerParams(
            dimension_semantics=("arbitrary",)),
    )(ps, pd, *attn_w)


def _silu(x):
    return x * jax.nn.sigmoid(x)


def _lnorm(x):
    m = x.mean(-1, keepdims=True)
    v = ((x - m) ** 2).mean(-1, keepdims=True)
    return (x - m) * lax.rsqrt(v + 1e-6)


def _rbf(d):
    c = lax.broadcasted_iota(jnp.int32, (1, _L), 1).astype(jnp.float32) * (
        10.0 / (_L - 1))
    w = 0.5 * 10.0 / _L
    return jnp.exp(-((d - c) ** 2) / (2.0 * w * w))


def _dot(a, b):
    return jnp.dot(a, b, preferred_element_type=jnp.float32)


def _attn_tail(elen, wa1, ba1, wa2, ba2, wa3, ba3):
    a = _silu(_lnorm(_dot(elen, wa1) + ba1))
    a = _silu(_lnorm(_dot(a, wa2) + ba2))
    return _dot(a, wa3) + ba3


def _masked_ex(elen, attn_refs):
    wa1, ba1, wa2, ba2, wa3, ba3 = (r[...] for r in attn_refs)
    logits = _attn_tail(elen, wa1, ba1, wa2, ba2, wa3, ba3)
    lane = lax.broadcasted_iota(jnp.int32, logits.shape, 1)
    return jnp.where(lane < _H, jnp.exp(logits), 0.0)


def _edge_body(has_next, *args):
    (gs_ref, gd_ref, ess_ref, giv_ref,
     wps_ref, wpd_ref, wpe_ref, wm_ref, wsp_ref,
     wg1_ref, bg1_ref, wg2_ref, bg2_ref,
     wa1_ref, ba1_ref, wa2_ref, ba2_ref, wa3_ref, ba3_ref,
     expand_ref, wl_ref) = args[:21]
    rest = args[21:]
    ess = ess_ref[...]
    gd = gd_ref[...]
    elen = _rbf(ess[:, _SH:_SH + 1])

    logits = _attn_tail(elen, wa1_ref[...], ba1_ref[...], wa2_ref[...],
                        ba2_ref[...], wa3_ref[...], ba3_ref[...])
    alpha = jnp.exp(logits) * giv_ref[:, :8]

    msg = (_dot(gs_ref[...], wps_ref[...])
           + _dot(gd, wpd_ref[...])
           + _dot(elen, wpe_ref[...]))
    gate = _dot(_silu(_dot(elen, wg1_ref[...]) + bg1_ref[...]),
                wg2_ref[...]) + bg2_ref[...]
    val = _dot(msg, wm_ref[...]) * _dot(ess, wsp_ref[...]) * gate
    if has_next:
        nattn, eo_ref, exn_ref = rest[:6], rest[6], rest[7]
        exn_ref[...] = _masked_ex(elen, nattn)
    else:
        eo_ref = rest[0]
    eo_ref[...] = _dot(val * _dot(alpha, expand_ref[...]), wl_ref[...])


def _bcast(shape):
    nd = len(shape)
    return pl.BlockSpec(shape, lambda i: (0,) * nd)


def _edge_call(gs, gd, ess, giv, weights, doutp, next_attn=None):
    grid = _EP // _BE
    has_next = next_attn is not None
    extra = tuple(next_attn) if has_next else ()
    out_specs = pl.BlockSpec((_BE, doutp), lambda i: (i, 0))
    out_shape = jax.ShapeDtypeStruct((_EP, doutp), jnp.float32)
    if has_next:
        out_specs = (out_specs, pl.BlockSpec((_BE, 8), lambda i: (i, 0)))
        out_shape = (out_shape, jax.ShapeDtypeStruct((_EP, 8), jnp.float32))
    return pl.pallas_call(
        functools.partial(_edge_body, has_next),
        grid=(grid,),
        in_specs=[
            pl.BlockSpec((_BE, _DP), lambda i: (i, 0)),
            pl.BlockSpec((_BE, _DP), lambda i: (i, 0)),
            pl.BlockSpec((_BE, 16), lambda i: (i, 0)),
            pl.BlockSpec((_BE, 16), lambda i: (i, 0)),
        ] + [_bcast(w.shape) for w in weights + extra],
        out_specs=out_specs,
        out_shape=out_shape,
        compiler_params=pltpu.CompilerParams(
            dimension_semantics=("arbitrary",)),
    )(gs, gd, ess, giv, *weights, *extra)


def _pad2(a, rows, cols):
    return jnp.zeros((rows, cols), jnp.float32).at[:a.shape[0], :a.shape[1]].set(a)


def _layer_weights(b, dout):
    doutp = 128 if dout <= 128 else dout
    wpre = b['W_pre']
    wps = _pad2(wpre[:_D], _DP, _DP)
    wpd = _pad2(wpre[_D:2 * _D], _DP, _DP)
    wpe = _pad2(wpre[2 * _D:], _L, _DP)
    wm = _pad2(b['Wm'], _DP, _VP)
    wsp = _pad2(b['Ws'], 16, _VP)
    wg1 = b['Wg1']
    bg1 = b['bg1'][None, :]
    wg2 = _pad2(b['Wg2'], _L, _VP)
    bg2 = _pad2(b['bg2'][None, :], 1, _VP)
    wa1 = b['Wa1']
    ba1 = b['ba1'][None, :]
    wa2 = b['Wa2']
    ba2 = b['ba2'][None, :]
    wa3 = _pad2(b['Wa3'], _L, 8)
    ba3 = _pad2(b['ba3'][None, :], 1, 8)
    expand = np.zeros((8, _VP), np.float32)
    for h in range(_H):
        expand[h, h * _D:(h + 1) * _D] = 1.0
    expand = jnp.asarray(expand)
    wl = _pad2(b['W_lin'][:_H * _D], _VP, doutp)
    wl_bot = _pad2(b['W_lin'][_H * _D:], _DP, doutp)
    attn_w = (wa1, ba1, wa2, ba2, wa3, ba3)
    edge_w = (wps, wpd, wpe, wm, wsp, wg1, bg1, wg2, bg2,
              wa1, ba1, wa2, ba2, wa3, ba3, expand, wl)
    return attn_w, edge_w, wl_bot, doutp


def kernel(pos, params, node_atom, edge_src, edge_dst, batch):
    p = params
    node = _dot(p['emb_table'][node_atom], p['W_to_irreps'])

    src = edge_src.astype(jnp.int32)
    dst = edge_dst.astype(jnp.int32)
    pad = _EP - _E
    src_p = jnp.concatenate([src, jnp.zeros((pad,), jnp.int32)])
    dst_p = jnp.concatenate([dst, jnp.full((pad,), _N, jnp.int32)])

    pos_t = jnp.zeros((_N + 1, 16), jnp.float32).at[:_N, :3].set(pos)
    ps, pd = _sc_gather(pos_t, src_p, dst_p, width=16, tc_tiling=False)

    attn_w1, edge_w1, wl_bot1, doutp1 = _layer_weights(p['blocks'][0], _D)
    attn_w2, edge_w2, wl_bot2, doutp2 = _layer_weights(p['blocks'][1], _F)

    ess, ex1 = _geom_call(ps, pd, attn_w1)

    nodep = jnp.zeros((_N + 1, _DP), jnp.float32).at[:_N, :_D].set(node)

    def inv_gather(ex):
        dn = jax.ops.segment_sum(ex, dst_p, num_segments=_N + 1)
        inv = 1.0 / (dn[:_N, :_H] + 1e-12)
        giv_t = jnp.zeros((_N + 1, 16), jnp.float32).at[:_N, :_H].set(inv)
        return _sc_gather1(giv_t, dst_p)

    giv = inv_gather(ex1)
    gs, gd = _sc_gather(nodep, src_p, dst_p)
    eo, ex2 = _edge_call(gs, gd, ess, giv, edge_w1, doutp1, next_attn=attn_w2)
    contrib = jax.ops.segment_sum(eo, dst_p, num_segments=_N + 1)
    nodep = contrib + _dot(nodep, wl_bot1)

    giv = inv_gather(ex2)
    gs, gd = _sc_gather(nodep, src_p, dst_p)
    eo = _edge_call(gs, gd, ess, giv, edge_w2, doutp2)
    contrib = jax.ops.segment_sum(eo, dst_p, num_segments=_N + 1)
    node2 = contrib + _dot(nodep, wl_bot2)

    h = _silu(_dot(node2[:_N], p['Wh1']) + p['bh1'])
    h = _silu(_dot(h, p['Wh2']) + p['bh2'])
    ne = _dot(h, p['Wh3']) + p['bh3']
    return jax.ops.segment_sum(ne, batch, num_segments=_G)


# speedup vs baseline: 2.0034x; 1.1124x over previous
import functools

import jax
import jax.numpy as jnp
import numpy as np
from jax import lax
from jax.experimental import pallas as pl
from jax.experimental.pallas import tpu as pltpu
from jax.experimental.pallas import tpu_sc as plsc

_N = 10000
_E = 160000
_G = 16
_D = 120
_H = 4
_L = 64
_SH = 9
_F = 256

_BE = 2048
_EP = 80 * _BE
_DP = 128
_VP = 512


_NC, _NS = 2, 16
_NW = _NC * _NS
_EW = _EP // _NW
_CG = 128


def _gather_body(node_hbm, src_hbm, dst_hbm,
                 gs_hbm, gd_hbm,
                 idx_s0, idx_d0, idx_s1, idx_d1,
                 rows_s, rows_d, sem_g0, sem_g1):
    wid = lax.axis_index("s") * _NC + lax.axis_index("c")
    base = wid * _EW
    nchunk = _EW // _CG
    sems = (sem_g0, sem_g1)
    idx_s = (idx_s0, idx_s1)
    idx_d = (idx_d0, idx_d1)

    def start(c, b):
        off = base + c * _CG
        pltpu.sync_copy(src_hbm.at[pl.ds(off, _CG)], idx_s[b])
        pltpu.sync_copy(dst_hbm.at[pl.ds(off, _CG)], idx_d[b])
        pltpu.async_copy(node_hbm.at[idx_s[b]], rows_s.at[b], sems[b])
        pltpu.async_copy(node_hbm.at[idx_d[b]], rows_d.at[b], sems[b])

    def finish(c, b):
        off = base + c * _CG
        pltpu.make_async_copy(node_hbm.at[idx_s[b]], rows_s.at[b],
                              sems[b]).wait()
        pltpu.make_async_copy(node_hbm.at[idx_d[b]], rows_d.at[b],
                              sems[b]).wait()
        pltpu.sync_copy(rows_s.at[b], gs_hbm.at[pl.ds(off, _CG)])
        pltpu.sync_copy(rows_d.at[b], gd_hbm.at[pl.ds(off, _CG)])

    start(0, 0)

    def step(j, carry):
        c0, c1 = 2 * j, 2 * j + 1
        start(c1, 1)
        finish(c0, 0)

        @pl.when(j < nchunk // 2 - 1)
        def _():
            start(c0 + 2, 0)

        finish(c1, 1)
        return carry

    lax.fori_loop(0, nchunk // 2, step, 0)


def _sc_gather(node_t, src_p, dst_p, width=_DP, tc_tiling=True,
               dtype=jnp.float32):
    f = pl.kernel(
        _gather_body,
        out_type=(jax.ShapeDtypeStruct((_EP, width), dtype),
                  jax.ShapeDtypeStruct((_EP, width), dtype)),
        mesh=plsc.VectorSubcoreMesh(core_axis_name="c", subcore_axis_name="s"),
        scratch_types=[
            pltpu.VMEM((_CG,), jnp.int32),
            pltpu.VMEM((_CG,), jnp.int32),
            pltpu.VMEM((_CG,), jnp.int32),
            pltpu.VMEM((_CG,), jnp.int32),
            pltpu.VMEM((2, _CG, width), dtype),
            pltpu.VMEM((2, _CG, width), dtype),
            pltpu.SemaphoreType.DMA,
            pltpu.SemaphoreType.DMA,
        ],
        compiler_params=pltpu.CompilerParams(use_tc_tiling_on_sc=tc_tiling),
    )
    return f(node_t, src_p, dst_p)


_NP4 = 16 * -(-((_N + 1) * _H) // 16)


def _dn_body(ex_hbm, dst_hbm, zed_hbm, out_hbm,
             acc, exb, idxb, sem):
    wid = lax.axis_index("s") * _NC + lax.axis_index("c")
    base = wid * _EW

    pltpu.sync_copy(zed_hbm, acc)
    pltpu.sync_copy(dst_hbm.at[pl.ds(base, _EW)], idxb)
    pltpu.async_copy(ex_hbm.at[pl.ds(base, _EW)], exb, sem).wait()

    lane = lax.iota(jnp.int32, 16)
    m = lane < _H
    zv = jnp.zeros((16,), jnp.int32)

    def it(e, carry):
        ev = jnp.full((16,), e, jnp.int32)
        dstv = plsc.load_gather(idxb, [ev])
        vals = plsc.load_gather(exb, [ev, lane], mask=m)
        plsc.addupdate_scatter(acc, [zv, dstv * _H + lane], vals, mask=m)
        return carry

    lax.fori_loop(0, _EW, it, 0)
    pltpu.sync_copy(acc, out_hbm.at[pl.ds(wid, 1)])


def _sc_dn(ex, dst_p):
    zed = jnp.zeros((1, _NP4), jnp.float32)
    f = pl.kernel(
        _dn_body,
        out_type=jax.ShapeDtypeStruct((_NW, _NP4), jnp.float32),
        mesh=plsc.VectorSubcoreMesh(core_axis_name="c", subcore_axis_name="s"),
        scratch_types=[
            pltpu.VMEM((1, _NP4), jnp.float32),
            pltpu.VMEM((_EW, 8), jnp.float32),
            pltpu.VMEM((_EW,), jnp.int32),
            pltpu.SemaphoreType.DMA,
        ],
        compiler_params=pltpu.CompilerParams(needs_layout_passes=False,
                                             use_tc_tiling_on_sc=False),
    )
    return f(ex, dst_p, zed)


def _gather1_body(tab_hbm, idx_hbm, out_hbm,
                  i0, i1, rows, sem0, sem1):
    wid = lax.axis_index("s") * _NC + lax.axis_index("c")
    base = wid * _EW
    nchunk = _EW // _CG
    sems = (sem0, sem1)
    idxs = (i0, i1)

    def start(c, b):
        off = base + c * _CG
        pltpu.sync_copy(idx_hbm.at[pl.ds(off, _CG)], idxs[b])
        pltpu.async_copy(tab_hbm.at[idxs[b]], rows.at[b], sems[b])

    def finish(c, b):
        off = base + c * _CG
        pltpu.make_async_copy(tab_hbm.at[idxs[b]], rows.at[b],
                              sems[b]).wait()
        pltpu.sync_copy(rows.at[b], out_hbm.at[pl.ds(off, _CG)])

    start(0, 0)

    def step(j, carry):
        c0, c1 = 2 * j, 2 * j + 1
        start(c1, 1)
        finish(c0, 0)

        @pl.when(j < nchunk // 2 - 1)
        def _():
            start(c0 + 2, 0)

        finish(c1, 1)
        return carry

    lax.fori_loop(0, nchunk // 2, step, 0)


def _sc_gather1(tab, idx, width=16):
    f = pl.kernel(
        _gather1_body,
        out_type=jax.ShapeDtypeStruct((_EP, width), jnp.float32),
        mesh=plsc.VectorSubcoreMesh(core_axis_name="c", subcore_axis_name="s"),
        scratch_types=[
            pltpu.VMEM((_CG,), jnp.int32),
            pltpu.VMEM((_CG,), jnp.int32),
            pltpu.VMEM((2, _CG, width), jnp.float32),
            pltpu.SemaphoreType.DMA,
            pltpu.SemaphoreType.DMA,
        ],
        compiler_params=pltpu.CompilerParams(use_tc_tiling_on_sc=False),
    )
    return f(tab, idx)


def _geom_body(ps_ref, pd_ref, wa1_ref, ba1_ref, wa2_ref, ba2_ref,
               wa3_ref, ba3_ref, ess_ref, ex_ref):
    diff = pd_ref[...] - ps_ref[...]
    d2 = jnp.sum(diff * diff, axis=-1, keepdims=True)
    dd = jnp.sqrt(d2)
    ex_ref[...] = _masked_ex(
        _rbf(dd), (wa1_ref, ba1_ref, wa2_ref, ba2_ref, wa3_ref, ba3_ref))
    inv = 1.0 / (dd + 1e-9)
    x = diff[:, 0:1] * inv
    y = diff[:, 1:2] * inv
    z = diff[:, 2:3] * inv
    s3, s15, s5 = np.sqrt(3.0), np.sqrt(15.0), np.sqrt(5.0)
    one = jnp.ones_like(x)
    ess_ref[...] = jnp.concatenate([
        one, s3 * x, s3 * y, s3 * z,
        s15 * x * y, s15 * y * z,
        (s5 / 2.0) * (2.0 * z * z - x * x - y * y),
        s15 * x * z, (s15 / 2.0) * (x * x - y * y),
        dd, jnp.zeros_like(x), jnp.zeros_like(x), jnp.zeros_like(x),
        jnp.zeros_like(x), jnp.zeros_like(x), jnp.zeros_like(x),
    ], axis=1)


def _geom_call(ps, pd, attn_w):
    grid = _EP // _BE
    return pl.pallas_call(
        _geom_body,
        grid=(grid,),
        in_specs=[pl.BlockSpec((_BE, 16), lambda i: (i, 0)),
                  pl.BlockSpec((_BE, 16), lambda i: (i, 0))]
        + [_bcast(w.shape) for w in attn_w],
        out_specs=(pl.BlockSpec((_BE, 16), lambda i: (i, 0)),
                   pl.BlockSpec((_BE, 8), lambda i: (i, 0))),
        out_shape=(jax.ShapeDtypeStruct((_EP, 16), jnp.float32),
                   jax.ShapeDtypeStruct((_EP, 8), jnp.float32)),
        compiler_params=pltpu.CompilerParams(
            dimension_semantics=("arbitrary",)),
    )(ps, pd, *attn_w)


def _silu(x):
    return x * jax.nn.sigmoid(x)


def _lnorm(x):
    m = x.mean(-1, keepdims=True)
    v = ((x - m) ** 2).mean(-1, keepdims=True)
    return (x - m) * lax.rsqrt(v + 1e-6)


def _rbf(d):
    c = lax.broadcasted_iota(jnp.int32, (1, _L), 1).astype(jnp.float32) * (
        10.0 / (_L - 1))
    w = 0.5 * 10.0 / _L
    return jnp.exp(-((d - c) ** 2) / (2.0 * w * w))


def _dot(a, b):
    return jnp.dot(a, b, preferred_element_type=jnp.float32)


def _attn_tail(elen, wa1, ba1, wa2, ba2, wa3, ba3):
    a = _silu(_lnorm(_dot(elen, wa1) + ba1))
    a = _silu(_lnorm(_dot(a, wa2) + ba2))
    return _dot(a, wa3) + ba3


def _masked_ex(elen, attn_refs):
    wa1, ba1, wa2, ba2, wa3, ba3 = (r[...] for r in attn_refs)
    logits = _attn_tail(elen, wa1, ba1, wa2, ba2, wa3, ba3)
    lane = lax.broadcasted_iota(jnp.int32, logits.shape, 1)
    return jnp.where(lane < _H, jnp.exp(logits), 0.0)


def _edge_body(has_next, *args):
    (gs_ref, gd_ref, ess_ref, giv_ref,
     wps_ref, wpd_ref, wpe_ref, wm_ref, wsp_ref,
     wg1_ref, bg1_ref, wg2_ref, bg2_ref,
     wa1_ref, ba1_ref, wa2_ref, ba2_ref, wa3_ref, ba3_ref,
     expand_ref, wl_ref) = args[:21]
    rest = args[21:]
    ess = ess_ref[...]
    gd = gd_ref[...]
    elen = _rbf(ess[:, _SH:_SH + 1])

    logits = _attn_tail(elen, wa1_ref[...], ba1_ref[...], wa2_ref[...],
                        ba2_ref[...], wa3_ref[...], ba3_ref[...])
    alpha = jnp.exp(logits) * giv_ref[:, :8]

    msg = (_dot(gs_ref[...], wps_ref[...])
           + _dot(gd, wpd_ref[...])
           + _dot(elen, wpe_ref[...]))
    gate = _dot(_silu(_dot(elen, wg1_ref[...]) + bg1_ref[...]),
                wg2_ref[...]) + bg2_ref[...]
    val = _dot(msg, wm_ref[...]) * _dot(ess, wsp_ref[...]) * gate
    if has_next:
        nattn, eo_ref, exn_ref = rest[:6], rest[6], rest[7]
        exn_ref[...] = _masked_ex(elen, nattn)
    else:
        eo_ref = rest[0]
    eo_ref[...] = _dot(val * _dot(alpha, expand_ref[...]), wl_ref[...])


def _bcast(shape):
    nd = len(shape)
    return pl.BlockSpec(shape, lambda i: (0,) * nd)


def _edge_call(gs, gd, ess, giv, weights, doutp, next_attn=None):
    grid = _EP // _BE
    has_next = next_attn is not None
    extra = tuple(next_attn) if has_next else ()
    out_specs = pl.BlockSpec((_BE, doutp), lambda i: (i, 0))
    out_shape = jax.ShapeDtypeStruct((_EP, doutp), jnp.float32)
    if has_next:
        out_specs = (out_specs, pl.BlockSpec((_BE, 8), lambda i: (i, 0)))
        out_shape = (out_shape, jax.ShapeDtypeStruct((_EP, 8), jnp.float32))
    return pl.pallas_call(
        functools.partial(_edge_body, has_next),
        grid=(grid,),
        in_specs=[
            pl.BlockSpec((_BE, _DP), lambda i: (i, 0)),
            pl.BlockSpec((_BE, _DP), lambda i: (i, 0)),
            pl.BlockSpec((_BE, 16), lambda i: (i, 0)),
            pl.BlockSpec((_BE, 16), lambda i: (i, 0)),
        ] + [_bcast(w.shape) for w in weights + extra],
        out_specs=out_specs,
        out_shape=out_shape,
        compiler_params=pltpu.CompilerParams(
            dimension_semantics=("arbitrary",)),
    )(gs, gd, ess, giv, *weights, *extra)


def _pad2(a, rows, cols):
    return jnp.zeros((rows, cols), jnp.float32).at[:a.shape[0], :a.shape[1]].set(a)


def _layer_weights(b, dout):
    doutp = 128 if dout <= 128 else dout
    wpre = b['W_pre']
    wps = _pad2(wpre[:_D], _DP, _DP)
    wpd = _pad2(wpre[_D:2 * _D], _DP, _DP)
    wpe = _pad2(wpre[2 * _D:], _L, _DP)
    wm = _pad2(b['Wm'], _DP, _VP)
    wsp = _pad2(b['Ws'], 16, _VP)
    wg1 = b['Wg1']
    bg1 = b['bg1'][None, :]
    wg2 = _pad2(b['Wg2'], _L, _VP)
    bg2 = _pad2(b['bg2'][None, :], 1, _VP)
    wa1 = b['Wa1']
    ba1 = b['ba1'][None, :]
    wa2 = b['Wa2']
    ba2 = b['ba2'][None, :]
    wa3 = _pad2(b['Wa3'], _L, 8)
    ba3 = _pad2(b['ba3'][None, :], 1, 8)
    expand = np.zeros((8, _VP), np.float32)
    for h in range(_H):
        expand[h, h * _D:(h + 1) * _D] = 1.0
    expand = jnp.asarray(expand)
    wl = _pad2(b['W_lin'][:_H * _D], _VP, doutp)
    wl_bot = _pad2(b['W_lin'][_H * _D:], _DP, doutp)
    attn_w = (wa1, ba1, wa2, ba2, wa3, ba3)
    edge_w = (wps, wpd, wpe, wm, wsp, wg1, bg1, wg2, bg2,
              wa1, ba1, wa2, ba2, wa3, ba3, expand, wl)
    return attn_w, edge_w, wl_bot, doutp


def kernel(pos, params, node_atom, edge_src, edge_dst, batch):
    p = params
    node = _dot(p['emb_table'][node_atom], p['W_to_irreps'])

    src = edge_src.astype(jnp.int32)
    dst = edge_dst.astype(jnp.int32)
    pad = _EP - _E
    src_p = jnp.concatenate([src, jnp.zeros((pad,), jnp.int32)])
    dst_p = jnp.concatenate([dst, jnp.full((pad,), _N, jnp.int32)])

    pos_t = jnp.zeros((_N + 1, 16), jnp.float32).at[:_N, :3].set(pos)
    ps, pd = _sc_gather(pos_t, src_p, dst_p, width=16, tc_tiling=False)

    attn_w1, edge_w1, wl_bot1, doutp1 = _layer_weights(p['blocks'][0], _D)
    attn_w2, edge_w2, wl_bot2, doutp2 = _layer_weights(p['blocks'][1], _F)

    ess, ex1 = _geom_call(ps, pd, attn_w1)

    nodep = jnp.zeros((_N + 1, _DP), jnp.float32).at[:_N, :_D].set(node)

    def inv_gather(ex):
        parts = _sc_dn(ex, dst_p)
        dn = parts.sum(0)[:(_N + 1) * _H].reshape(_N + 1, _H)
        inv = 1.0 / (dn[:_N] + 1e-12)
        giv_t = jnp.zeros((_N + 1, 16), jnp.float32).at[:_N, :_H].set(inv)
        return _sc_gather1(giv_t, dst_p)

    giv = inv_gather(ex1)
    gs, gd = _sc_gather(nodep, src_p, dst_p)
    eo, ex2 = _edge_call(gs, gd, ess, giv, edge_w1, doutp1, next_attn=attn_w2)
    contrib = jax.ops.segment_sum(eo, dst_p, num_segments=_N + 1)
    nodep = contrib + _dot(nodep, wl_bot1)

    giv = inv_gather(ex2)
    gs, gd = _sc_gather(nodep, src_p, dst_p)
    eo = _edge_call(gs, gd, ess, giv, edge_w2, doutp2)
    contrib = jax.ops.segment_sum(eo, dst_p, num_segments=_N + 1)
    node2 = contrib + _dot(nodep, wl_bot2)

    h = _silu(_dot(node2[:_N], p['Wh1']) + p['bh1'])
    h = _silu(_dot(h, p['Wh2']) + p['bh2'])
    ne = _dot(h, p['Wh3']) + p['bh3']
    return jax.ops.segment_sum(ne, batch, num_segments=_G)
